# baseline (speedup 1.0000x reference)
import functools

import jax
import jax.numpy as jnp
from jax import lax
from jax.experimental import pallas as pl
from jax.experimental.pallas import tpu as pltpu
from jax.experimental.pallas import tpu_sc as plsc

SIGMA = 1.0

_PB = 128
_K = 32
_CH = 128
_NBUF = 4


def _sc_gather(T, nidx):
    E = nidx.shape[0]
    info = plsc.get_sparse_core_info()
    nw = info.num_cores * info.num_subcores
    epw = E // nw
    nit = epw // (_CH * _NBUF)
    mesh = plsc.VectorSubcoreMesh(core_axis_name="c", subcore_axis_name="s")

    @functools.partial(
        pl.kernel,
        out_type=jax.ShapeDtypeStruct((E, 128), jnp.int32),
        mesh=mesh,
        scratch_types=[
            [pltpu.VMEM((_CH,), jnp.int32) for _ in range(_NBUF)],
            pltpu.VMEM((_CH * _NBUF, 128), jnp.int32),
            pltpu.SemaphoreType.DMA,
            pltpu.SemaphoreType.DMA,
        ],
    )
    def gather_k(t_hbm, idx_hbm, out_hbm, idxs, rows_v, si, sg):
        wid = lax.axis_index("s") * info.num_cores + lax.axis_index("c")
        base = wid * epw

        def body(j, carry):
            off = base + j * (_CH * _NBUF)
            idx_cps = [
                pltpu.async_copy(
                    idx_hbm.at[pl.ds(off + b * _CH, _CH)], idxs[b], si)
                for b in range(_NBUF)
            ]
            for b in range(_NBUF):
                idx_cps[b].wait()
            gathers = [
                pltpu.async_copy(
                    t_hbm.at[idxs[b]], rows_v.at[pl.ds(b * _CH, _CH)], sg)
                for b in range(_NBUF)
            ]
            for b in range(_NBUF):
                gathers[b].wait()
            pltpu.sync_copy(rows_v, out_hbm.at[pl.ds(off, _CH * _NBUF)])
            return carry

        lax.fori_loop(0, nit, body, 0)

    return gather_k(T, nidx)


def _tc_body(nfp_ref, xp_ref, aux_ref, wcat_ref, whi_ref, out_ref):
    pb = out_ref.shape[0]
    u = nfp_ref[...]
    xp = xp_ref[...]
    eb = pb * _K

    nf = lax.bitcast_convert_type(
        u & jnp.int32(-65536), jnp.float32).astype(jnp.bfloat16)

    xw = lax.bitcast_convert_type(lax.shift_left(u[:, :6], 16), jnp.float32)
    nx3 = xw[:, :3] + xw[:, 3:6]
    nx16 = jnp.concatenate(
        [nx3, jnp.zeros((eb, 13), jnp.float32)], axis=1)

    rel = (nx16.reshape(pb, _K, 16) - xp[:, None, :]).reshape(eb, 16)
    sq_rel = jnp.sum(rel * rel, axis=-1, keepdims=True)
    relkp = jax.lax.dot(rel, aux_ref[...],
                        precision=jax.lax.Precision.HIGHEST,
                        preferred_element_type=jnp.float32)
    sq_d = sq_rel - 2.0 * relkp[:, :15] + aux_ref[3:4, :15]
    dist = jnp.sqrt(jnp.maximum(sq_d, 0.0) + 1e-12)
    h = jnp.maximum(0.0, 1.0 - dist / SIGMA)

    h3 = h.astype(jnp.bfloat16).reshape(pb, _K, 15)
    nf3 = nf.reshape(pb, _K, 128)
    weighted = jax.lax.dot_general(
        h3, nf3, (((1,), (1,)), ((0,), (0,))),
        preferred_element_type=jnp.float32)

    whi = weighted.astype(jnp.bfloat16)
    wres = (weighted - whi.astype(jnp.float32)).astype(jnp.bfloat16)

    acc = jnp.zeros((pb, 128), jnp.float32)
    for k in range(15):
        t = jax.lax.dot(whi[:, k, :], wcat_ref[k],
                        preferred_element_type=jnp.float32)
        acc = acc + t[:, :128] + t[:, 128:]
        acc = acc + jax.lax.dot(wres[:, k, :], whi_ref[k],
                                preferred_element_type=jnp.float32)
    out_ref[...] = acc


def _tc_compute(NFP, Xp, aux, Wcat, Whi, interpret=False):
    E = NFP.shape[0]
    npts = E // _K
    grid = (npts // _PB,)
    eb = _PB * _K
    return pl.pallas_call(
        _tc_body,
        grid=grid,
        in_specs=[
            pl.BlockSpec((eb, 128), lambda b: (b, 0)),
            pl.BlockSpec((_PB, 16), lambda b: (b, 0)),
            pl.BlockSpec((16, 16), lambda b: (0, 0)),
            pl.BlockSpec((15, 128, 256), lambda b: (0, 0, 0)),
            pl.BlockSpec((15, 128, 128), lambda b: (0, 0, 0)),
        ],
        out_specs=pl.BlockSpec((_PB, 128), lambda b: (b, 0)),
        out_shape=jax.ShapeDtypeStruct((npts, 128), jnp.float32),
        interpret=interpret,
    )(NFP, Xp, aux, Wcat, Whi)


def _build_table(X, F):
    n = X.shape[0]

    def b16(v):
        return lax.bitcast_convert_type(
            v.astype(jnp.bfloat16), jnp.uint16).astype(jnp.uint32)

    fhi = b16(F) << 16
    xhi = X.astype(jnp.bfloat16)
    xlo = X - xhi.astype(jnp.float32)
    low = jnp.concatenate(
        [b16(X), b16(xlo), jnp.zeros((n, 122), jnp.uint32)], axis=1)
    return lax.bitcast_convert_type(fhi | low, jnp.int32)


def kernel(X, F, N, kernel_points, W):
    n = X.shape[0]
    align = _CH * _NBUF
    assert align % _PB == 0
    npad = ((n + align - 1) // align) * align

    Xp = jnp.pad(X, ((0, npad - n), (0, 16 - X.shape[1])))
    Npad = jnp.pad(N, ((0, npad - n), (0, 0)))
    nidx = Npad.reshape(-1)

    aux = jnp.zeros((16, 16), jnp.float32)
    aux = aux.at[:3, :15].set(kernel_points.T)
    aux = aux.at[3, :15].set(jnp.sum(kernel_points * kernel_points, axis=-1))

    Whi = W.astype(jnp.bfloat16)
    Wlo = (W - Whi.astype(jnp.float32)).astype(jnp.bfloat16)
    Wcat = jnp.concatenate([Whi, Wlo], axis=2)

    T = _build_table(X, F)
    NFP = _sc_gather(T, nidx)
    out = _tc_compute(NFP, Xp, aux, Wcat, Whi)
    return out[:n]


# speedup vs baseline: 1.2252x; 1.2252x over previous
import functools

import jax
import jax.numpy as jnp
from jax import lax
from jax.experimental import pallas as pl
from jax.experimental.pallas import tpu as pltpu
from jax.experimental.pallas import tpu_sc as plsc

SIGMA = 1.0

_PB = 128
_K = 32
_CH = 128


def _sc_gather(T, nidx):
    E = nidx.shape[0]
    info = plsc.get_sparse_core_info()
    nw = info.num_cores * info.num_subcores
    epw = E // nw
    nit = epw // _CH
    mesh = plsc.VectorSubcoreMesh(core_axis_name="c", subcore_axis_name="s")

    @functools.partial(
        pl.kernel,
        out_type=jax.ShapeDtypeStruct((E, 128), jnp.int32),
        mesh=mesh,
        scratch_types=[
            pltpu.VMEM((_CH,), jnp.int32),
            pltpu.VMEM((_CH, 128), jnp.int32),
            pltpu.SemaphoreType.DMA,
        ],
    )
    def gather_k(t_hbm, idx_hbm, out_hbm, idx_v, rows_v, sg):
        wid = lax.axis_index("s") * info.num_cores + lax.axis_index("c")
        base = wid * epw

        def body(j, carry):
            off = base + j * _CH
            pltpu.sync_copy(idx_hbm.at[pl.ds(off, _CH)], idx_v)
            pltpu.async_copy(t_hbm.at[idx_v], rows_v, sg).wait()
            pltpu.sync_copy(rows_v, out_hbm.at[pl.ds(off, _CH)])
            return carry

        lax.fori_loop(0, nit, body, 0)

    return gather_k(T, nidx)


def _tc_body(nfp_ref, xp_ref, aux_ref, wcat_ref, whi_ref, out_ref):
    pb = out_ref.shape[0]
    u = nfp_ref[...]
    xp = xp_ref[...]
    eb = pb * _K

    nf = lax.bitcast_convert_type(
        u & jnp.int32(-65536), jnp.float32).astype(jnp.bfloat16)

    xw = lax.bitcast_convert_type(lax.shift_left(u[:, :6], 16), jnp.float32)
    nx3 = xw[:, :3] + xw[:, 3:6]
    nx16 = jnp.concatenate(
        [nx3, jnp.zeros((eb, 13), jnp.float32)], axis=1)

    rel = (nx16.reshape(pb, _K, 16) - xp[:, None, :]).reshape(eb, 16)
    sq_rel = jnp.sum(rel * rel, axis=-1, keepdims=True)
    relkp = jax.lax.dot(rel, aux_ref[...],
                        precision=jax.lax.Precision.HIGHEST,
                        preferred_element_type=jnp.float32)
    sq_d = sq_rel - 2.0 * relkp[:, :15] + aux_ref[3:4, :15]
    dist = jnp.sqrt(jnp.maximum(sq_d, 0.0) + 1e-12)
    h = jnp.maximum(0.0, 1.0 - dist / SIGMA)

    h3 = h.astype(jnp.bfloat16).reshape(pb, _K, 15)
    nf3 = nf.reshape(pb, _K, 128)
    weighted = jax.lax.dot_general(
        h3, nf3, (((1,), (1,)), ((0,), (0,))),
        preferred_element_type=jnp.float32)

    whi = weighted.astype(jnp.bfloat16)
    wres = (weighted - whi.astype(jnp.float32)).astype(jnp.bfloat16)

    acc = jnp.zeros((pb, 128), jnp.float32)
    for k in range(15):
        t = jax.lax.dot(whi[:, k, :], wcat_ref[k],
                        preferred_element_type=jnp.float32)
        acc = acc + t[:, :128] + t[:, 128:]
        acc = acc + jax.lax.dot(wres[:, k, :], whi_ref[k],
                                preferred_element_type=jnp.float32)
    out_ref[...] = acc


def _tc_compute(NFP, Xp, aux, Wcat, Whi, interpret=False):
    E = NFP.shape[0]
    npts = E // _K
    grid = (npts // _PB,)
    eb = _PB * _K
    return pl.pallas_call(
        _tc_body,
        grid=grid,
        in_specs=[
            pl.BlockSpec((eb, 128), lambda b: (b, 0)),
            pl.BlockSpec((_PB, 16), lambda b: (b, 0)),
            pl.BlockSpec((16, 16), lambda b: (0, 0)),
            pl.BlockSpec((15, 128, 256), lambda b: (0, 0, 0)),
            pl.BlockSpec((15, 128, 128), lambda b: (0, 0, 0)),
        ],
        out_specs=pl.BlockSpec((_PB, 128), lambda b: (b, 0)),
        out_shape=jax.ShapeDtypeStruct((npts, 128), jnp.float32),
        interpret=interpret,
    )(NFP, Xp, aux, Wcat, Whi)


def _build_table(X, F):
    n = X.shape[0]

    def b16(v):
        return lax.bitcast_convert_type(
            v.astype(jnp.bfloat16), jnp.uint16).astype(jnp.uint32)

    fhi = b16(F) << 16
    xhi = X.astype(jnp.bfloat16)
    xlo = X - xhi.astype(jnp.float32)
    low = jnp.concatenate(
        [b16(X), b16(xlo), jnp.zeros((n, 122), jnp.uint32)], axis=1)
    return lax.bitcast_convert_type(fhi | low, jnp.int32)


def kernel(X, F, N, kernel_points, W):
    n = X.shape[0]
    assert _CH % _PB == 0 or _PB % _CH == 0
    align = max(_PB, _CH)
    npad = ((n + align - 1) // align) * align

    Xp = jnp.pad(X, ((0, npad - n), (0, 16 - X.shape[1])))
    Npad = jnp.pad(N, ((0, npad - n), (0, 0)))
    nidx = Npad.reshape(-1)

    aux = jnp.zeros((16, 16), jnp.float32)
    aux = aux.at[:3, :15].set(kernel_points.T)
    aux = aux.at[3, :15].set(jnp.sum(kernel_points * kernel_points, axis=-1))

    Whi = W.astype(jnp.bfloat16)
    Wlo = (W - Whi.astype(jnp.float32)).astype(jnp.bfloat16)
    Wcat = jnp.concatenate([Whi, Wlo], axis=2)

    T = _build_table(X, F)
    NFP = _sc_gather(T, nidx)
    out = _tc_compute(NFP, Xp, aux, Wcat, Whi)
    return out[:n]
